# baseline (speedup 1.0000x reference)
import jax
import jax.numpy as jnp
from jax import lax
from jax.experimental import pallas as pl
from jax.experimental.pallas import tpu as pltpu
from jax.experimental.pallas import tpu_sc as plsc

H = 32
W = 32
D2 = 384
L = 16
VECS = D2 // L
NC = 2
NS = 16


def _emb_body(row_hbm, col_hbm, out_hbm, rvec, cblk, oblk):
    c = lax.axis_index("c")
    s = lax.axis_index("s")
    w = s * NC + c

    pltpu.sync_copy(row_hbm.at[w], rvec)
    pltpu.sync_copy(col_hbm, cblk)

    rv = [rvec[pl.ds(k * L, L)] for k in range(VECS)]

    def body_j(j, carry):
        for k in range(VECS):
            oblk[j, pl.ds(k * L, L)] = rv[k]
            oblk[j, pl.ds(D2 + k * L, L)] = cblk[j, pl.ds(k * L, L)]
        return carry

    lax.fori_loop(0, W, body_j, 0)

    pltpu.sync_copy(oblk, out_hbm.at[pl.ds(w * W, W)])


def kernel(row_emb, col_emb, h, w):
    mesh = plsc.VectorSubcoreMesh(core_axis_name="c", subcore_axis_name="s")
    f = pl.kernel(
        _emb_body,
        mesh=mesh,
        out_type=jax.ShapeDtypeStruct((H * W, 2 * D2), jnp.float32),
        scratch_types=[
            pltpu.VMEM((D2,), jnp.float32),
            pltpu.VMEM((W, D2), jnp.float32),
            pltpu.VMEM((W, 2 * D2), jnp.float32),
        ],
    )
    return f(row_emb, col_emb)


# speedup vs baseline: 1.0876x; 1.0876x over previous
import jax
import jax.numpy as jnp
from jax import lax
from jax.experimental import pallas as pl
from jax.experimental.pallas import tpu as pltpu
from jax.experimental.pallas import tpu_sc as plsc

H = 32
W = 32
D2 = 384
L = 16
VECS = D2 // L
NC = 2
NS = 16


def _emb_body(row_hbm, col_hbm, out_hbm, rvec, oblk, csem):
    c = lax.axis_index("c")
    s = lax.axis_index("s")
    w = s * NC + c

    ccp = pltpu.async_copy(col_hbm, oblk.at[:, pl.ds(D2, D2)], csem)
    pltpu.sync_copy(row_hbm.at[w], rvec)

    rv = [rvec[pl.ds(k * L, L)] for k in range(VECS)]

    def body_j(j, carry):
        for k in range(VECS):
            oblk[j, pl.ds(k * L, L)] = rv[k]
        return carry

    lax.fori_loop(0, W, body_j, 0)

    ccp.wait()
    pltpu.sync_copy(oblk, out_hbm.at[pl.ds(w * W, W)])


def kernel(row_emb, col_emb, h, w):
    mesh = plsc.VectorSubcoreMesh(core_axis_name="c", subcore_axis_name="s")
    f = pl.kernel(
        _emb_body,
        mesh=mesh,
        out_type=jax.ShapeDtypeStruct((H * W, 2 * D2), jnp.float32),
        scratch_types=[
            pltpu.VMEM((D2,), jnp.float32),
            pltpu.VMEM((W, 2 * D2), jnp.float32),
            pltpu.SemaphoreType.DMA,
        ],
    )
    return f(row_emb, col_emb)
